# baseline (speedup 1.0000x reference)
import functools
import math

import jax
import jax.numpy as jnp
from jax import lax
from jax.experimental import pallas as pl
from jax.experimental.pallas import tpu as pltpu
from jax.experimental.pallas import tpu_sc as plsc

_SIGMA = 0.1
_SCALE = -0.5 / (_SIGMA * _SIGMA)
_NC, _NS, _L = 2, 16, 16
_NW = _NC * _NS


def _sc_partial(sample, loc, weight):
    B, N, D = loc.shape
    BPW = B // _NW
    CH = 2048
    NCH = N // CH
    G = CH // _L

    mesh = plsc.VectorSubcoreMesh(core_axis_name="c", subcore_axis_name="s")

    @functools.partial(
        pl.kernel,
        out_type=(jax.ShapeDtypeStruct((B, _L), jnp.float32),
                  jax.ShapeDtypeStruct((B, _L), jnp.float32)),
        mesh=mesh,
        compiler_params=pltpu.CompilerParams(needs_layout_passes=False,
                                             use_tc_tiling_on_sc=False),
        scratch_types=[
            pltpu.VMEM((CH, D), jnp.float32),
            pltpu.VMEM((CH, D), jnp.float32),
            pltpu.VMEM((CH,), jnp.float32),
            pltpu.VMEM((CH,), jnp.float32),
            pltpu.VMEM((_L,), jnp.float32),
            pltpu.VMEM((_L,), jnp.float32),
            pltpu.VMEM((_L,), jnp.float32),
            pltpu.SemaphoreType.DMA,
            pltpu.SemaphoreType.DMA,
            pltpu.SemaphoreType.DMA,
            pltpu.SemaphoreType.DMA,
        ],
    )
    def k(sample_hbm, loc_hbm, weight_hbm, m_hbm, s_hbm,
          lbuf0, lbuf1, wbuf0, wbuf1, sbuf, mstage, sstage,
          lsem0, lsem1, wsem0, wsem1):
        wid = lax.axis_index("s") * _NC + lax.axis_index("c")
        lbufs = (lbuf0, lbuf1)
        wbufs = (wbuf0, wbuf1)
        lsems = (lsem0, lsem1)
        wsems = (wsem0, wsem1)
        lane = lax.broadcasted_iota(jnp.int32, (_L,), 0)
        perms = [(lane + k) & (D - 1) for k in range(D)]
        for bb in range(BPW):
            b = wid * BPW + bb
            pltpu.sync_copy(sample_hbm.at[b], sbuf)
            sperm = [plsc.load_gather(sbuf, [perms[k]]) for k in range(D)]
            cps = {0: (pltpu.async_copy(loc_hbm.at[b, pl.ds(0, CH)],
                                        lbufs[0], lsems[0]),
                       pltpu.async_copy(weight_hbm.at[b, pl.ds(0, CH)],
                                        wbufs[0], wsems[0]))}
            m = jnp.full((_L,), -1e30, jnp.float32)
            s = jnp.zeros((_L,), jnp.float32)
            for c in range(NCH):
                if c + 1 < NCH:
                    sl = (c + 1) % 2
                    cps[c + 1] = (
                        pltpu.async_copy(loc_hbm.at[b, pl.ds((c + 1) * CH, CH)],
                                         lbufs[sl], lsems[sl]),
                        pltpu.async_copy(weight_hbm.at[b, pl.ds((c + 1) * CH, CH)],
                                         wbufs[sl], wsems[sl]))
                cps[c][0].wait()
                cps[c][1].wait()
                lb = lbufs[c % 2]
                wb = wbufs[c % 2]

                def body(g, carry, lb=lb, wb=wb):
                    m, s = carry
                    rows = g * _L + lane
                    acc = jnp.zeros((_L,), jnp.float32)
                    for kd in range(D):
                        v = plsc.load_gather(lb, [rows, perms[kd]])
                        t = v - sperm[kd]
                        acc = acc + t * t
                    w = wb[pl.ds(g * _L, _L)]
                    z = _SCALE * acc + w
                    mn = jnp.maximum(m, z)
                    s = s * jnp.exp(m - mn) + jnp.exp(z - mn)
                    return mn, s

                m, s = lax.fori_loop(0, G, body, (m, s))
            mstage[...] = m
            sstage[...] = s
            pltpu.sync_copy(mstage, m_hbm.at[b])
            pltpu.sync_copy(sstage, s_hbm.at[b])

    return k(sample, loc, weight)


def _tc_finish(m_arr, s_arr):
    B, L = m_arr.shape
    d = 16
    const = -0.5 * d * math.log(2.0 * math.pi * _SIGMA * _SIGMA)

    def fk(m_ref, s_ref, o_ref):
        m = m_ref[...]
        s = s_ref[...]
        mx = jnp.max(m, axis=1, keepdims=True)
        tot = jnp.sum(s * jnp.exp(m - mx), axis=1, keepdims=True)
        o_ref[...] = mx + jnp.log(tot) + const

    return pl.pallas_call(
        fk,
        out_shape=jax.ShapeDtypeStruct((B, 1), jnp.float32),
    )(m_arr, s_arr)


def kernel(sample, loc, weight):
    m_arr, s_arr = _sc_partial(sample, loc, weight)
    return _tc_finish(m_arr, s_arr).reshape(-1)


# speedup vs baseline: 8.4502x; 8.4502x over previous
import functools
import math

import jax
import jax.numpy as jnp
from jax import lax
from jax.experimental import pallas as pl
from jax.experimental.pallas import tpu as pltpu
from jax.experimental.pallas import tpu_sc as plsc

_SIGMA = 0.1
_SCALE = -0.5 / (_SIGMA * _SIGMA)
_NC, _NS, _L = 2, 16, 16
_NW = _NC * _NS


def _sc_partial(sample, loc_v, wq, B, N, D):
    BPW = B // _NW
    CHN = 2048
    NT = CHN // 128
    NCH = N // CHN

    mesh = plsc.VectorSubcoreMesh(core_axis_name="c", subcore_axis_name="s")

    @functools.partial(
        pl.kernel,
        out_type=(jax.ShapeDtypeStruct((B, _L), jnp.float32),
                  jax.ShapeDtypeStruct((B, _L), jnp.float32)),
        mesh=mesh,
        compiler_params=pltpu.CompilerParams(needs_layout_passes=False,
                                             use_tc_tiling_on_sc=False),
        scratch_types=[
            pltpu.VMEM((D // 8, NT, 8, 128), jnp.float32),
            pltpu.VMEM((D // 8, NT, 8, 128), jnp.float32),
            pltpu.VMEM((NT, 128), jnp.float32),
            pltpu.VMEM((NT, 128), jnp.float32),
            pltpu.VMEM((_L,), jnp.float32),
            pltpu.VMEM((_L,), jnp.float32),
            pltpu.VMEM((_L,), jnp.float32),
            pltpu.SemaphoreType.DMA,
            pltpu.SemaphoreType.DMA,
            pltpu.SemaphoreType.DMA,
            pltpu.SemaphoreType.DMA,
        ],
    )
    def k(sample_hbm, loc_hbm, w_hbm, m_hbm, s_hbm,
          lbuf0, lbuf1, wbuf0, wbuf1, sbuf, mstage, sstage,
          lsem0, lsem1, wsem0, wsem1):
        wid = lax.axis_index("s") * _NC + lax.axis_index("c")
        lbufs = (lbuf0, lbuf1)
        wbufs = (wbuf0, wbuf1)
        lsems = (lsem0, lsem1)
        wsems = (wsem0, wsem1)
        lane = lax.broadcasted_iota(jnp.int32, (_L,), 0)
        for bb in range(BPW):
            b = wid * BPW + bb
            wa = b // 8
            wcol = (b % 8) * 128
            pltpu.sync_copy(sample_hbm.at[b], sbuf)
            sv = sbuf[...]
            sd = [jnp.sum(jnp.where(lane == d, sv, 0.0)) for d in range(D)]

            def start(c):
                sl = c % 2
                return (
                    pltpu.async_copy(loc_hbm.at[b, :, pl.ds(c * NT, NT)],
                                     lbufs[sl], lsems[sl]),
                    pltpu.async_copy(w_hbm.at[wa, pl.ds(c * NT, NT),
                                              pl.ds(wcol, 128)],
                                     wbufs[sl], wsems[sl]))

            cps = {0: start(0)}
            m = jnp.full((_L,), -1e30, jnp.float32)
            s = jnp.zeros((_L,), jnp.float32)
            for c in range(NCH):
                if c + 1 < NCH:
                    cps[c + 1] = start(c + 1)
                cps[c][0].wait()
                cps[c][1].wait()
                lb = lbufs[c % 2]
                wb = wbufs[c % 2]

                def body(t, carry, lb=lb, wb=wb):
                    m, s = carry
                    for j in range(8):
                        acc = jnp.zeros((_L,), jnp.float32)
                        for dd in range(D // 8):
                            for ss in range(8):
                                v = lb[dd, t, ss, pl.ds(j * _L, _L)]
                                tt = v - sd[dd * 8 + ss]
                                acc = acc + tt * tt
                        w = wb[t, pl.ds(j * _L, _L)]
                        z = _SCALE * acc + w
                        mn = jnp.maximum(m, z)
                        s = s * jnp.exp(m - mn) + jnp.exp(z - mn)
                        m = mn
                    return m, s

                m, s = lax.fori_loop(0, NT, body, (m, s))
            mstage[...] = m
            sstage[...] = s
            pltpu.sync_copy(mstage, m_hbm.at[b])
            pltpu.sync_copy(sstage, s_hbm.at[b])

    return k(sample, loc_v, wq)


def _tc_finish(m_arr, s_arr):
    B, L = m_arr.shape
    d = 16
    const = -0.5 * d * math.log(2.0 * math.pi * _SIGMA * _SIGMA)

    def fk(m_ref, s_ref, o_ref):
        m = m_ref[...]
        s = s_ref[...]
        mx = jnp.max(m, axis=1, keepdims=True)
        tot = jnp.sum(s * jnp.exp(m - mx), axis=1, keepdims=True)
        o_ref[...] = mx + jnp.log(tot) + const

    return pl.pallas_call(
        fk,
        out_shape=jax.ShapeDtypeStruct((B, 1), jnp.float32),
    )(m_arr, s_arr)


def kernel(sample, loc, weight):
    B, N, D = loc.shape
    loc_v = loc.reshape(B, N // 128, 128, D // 8, 8).transpose(0, 3, 1, 4, 2)
    wq = weight.reshape(B // 8, 8, N // 128, 128).transpose(0, 2, 1, 3)
    wq = wq.reshape(B // 8, N // 128, 8 * 128)
    m_arr, s_arr = _sc_partial(sample, loc_v, wq, B, N, D)
    return _tc_finish(m_arr, s_arr).reshape(-1)
